# baseline (speedup 1.0000x reference)
import functools

import jax
import jax.numpy as jnp
from jax import lax
from jax.experimental import pallas as pl
from jax.experimental.pallas import tpu as pltpu
from jax.experimental.pallas import tpu_sc as plsc

N = 10000
D = 128
E = 320000
NC = 2
NS = 16
NW = NC * NS
CH = 80
E_T = E // NW
NCHUNK = E_T // CH
H0 = 64
H1 = NCHUNK - H0
ROWS_A = 624
TAIL_BASE = ROWS_A * NS
ZR = 32
ZREM = ROWS_A % ZR


def _sc_body(src_hbm, dst_hbm, x_hbm, out_hbm,
             src_v, dst_v, rows0, rows1, stage_v, agg_sh, sem0, sem1):
    cid = lax.axis_index("c")
    sid = lax.axis_index("s")

    z16 = jnp.zeros((16,), jnp.float32)

    @pl.loop(0, ZR)
    def _zero(i):
        for l in range(D // 16):
            stage_v[i, pl.ds(l * 16, 16)] = z16

    base = sid * ROWS_A

    @pl.loop(0, ROWS_A // ZR)
    def _zero_dma(i):
        pltpu.sync_copy(stage_v, agg_sh.at[pl.ds(base + i * ZR, ZR)])

    pltpu.sync_copy(stage_v.at[pl.ds(0, ZREM)],
                    agg_sh.at[pl.ds(base + (ROWS_A // ZR) * ZR, ZREM)])

    @pl.when(sid == NS - 1)
    def _zero_tail():
        pltpu.sync_copy(stage_v.at[pl.ds(0, N - TAIL_BASE)],
                        agg_sh.at[pl.ds(TAIL_BASE, N - TAIL_BASE)])

    pltpu.sync_copy(dst_hbm.at[cid, sid], dst_v)
    pltpu.sync_copy(src_hbm.at[cid, sid, pl.ds(0, H0)], src_v)

    plsc.subcore_barrier()

    def gather(c, off, rows, sem):
        return pltpu.async_copy(x_hbm.at[src_v.at[c - off]], rows, sem)

    def wait(c, off, rows, sem):
        pltpu.make_async_copy(x_hbm.at[src_v.at[c - off]], rows, sem).wait()

    def scatter(c, rows):
        pltpu.sync_copy(rows, agg_sh.at[dst_v.at[c]], add=True)

    gather(0, 0, rows0, sem0)

    @pl.loop(0, H0 - 2, step=2)
    def _edges0(c):
        gather(c + 1, 0, rows1, sem1)
        wait(c, 0, rows0, sem0)
        scatter(c, rows0)
        gather(c + 2, 0, rows0, sem0)
        wait(c + 1, 0, rows1, sem1)
        scatter(c + 1, rows1)

    gather(H0 - 1, 0, rows1, sem1)
    wait(H0 - 2, 0, rows0, sem0)
    scatter(H0 - 2, rows0)
    wait(H0 - 1, 0, rows1, sem1)
    scatter(H0 - 1, rows1)

    pltpu.sync_copy(src_hbm.at[cid, sid, pl.ds(H0, H1)],
                    src_v.at[pl.ds(0, H1)])
    gather(H0, H0, rows0, sem0)

    @pl.loop(H0, H0 + H1 - 1, step=2)
    def _edges1(c):
        gather(c + 1, H0, rows1, sem1)
        wait(c, H0, rows0, sem0)
        scatter(c, rows0)
        gather(c + 2, H0, rows0, sem0)
        wait(c + 1, H0, rows1, sem1)
        scatter(c + 1, rows1)

    wait(NCHUNK - 1, H0, rows0, sem0)
    scatter(NCHUNK - 1, rows0)

    plsc.subcore_barrier()

    sl = pl.ds(base, ROWS_A)
    pltpu.sync_copy(agg_sh.at[sl], out_hbm.at[cid].at[sl])

    @pl.when(sid == NS - 1)
    def _write_tail():
        tl = pl.ds(TAIL_BASE, N - TAIL_BASE)
        pltpu.sync_copy(agg_sh.at[tl], out_hbm.at[cid].at[tl])


_sc_scatter = functools.partial(
    pl.kernel,
    out_type=jax.ShapeDtypeStruct((NC, N, D), jnp.float32),
    mesh=plsc.VectorSubcoreMesh(core_axis_name="c", subcore_axis_name="s"),
    scratch_types=[
        pltpu.VMEM((H0, CH), jnp.int32),
        pltpu.VMEM((NCHUNK, CH), jnp.int32),
        pltpu.VMEM((CH, D), jnp.float32),
        pltpu.VMEM((CH, D), jnp.float32),
        pltpu.VMEM((ZR, D), jnp.float32),
        pltpu.VMEM_SHARED((N, D), jnp.float32),
        pltpu.SemaphoreType.DMA,
        pltpu.SemaphoreType.DMA,
    ],
)(_sc_body)


MM_BLK = 1000


def _mm_body(p_ref, w_ref, o_ref):
    acc = p_ref[0] + p_ref[1]
    o_ref[...] = lax.dot_general(
        acc, w_ref[...], (((1,), (1,)), ((), ())),
        preferred_element_type=jnp.float32)


def _tc_matmul(partials, W):
    return pl.pallas_call(
        _mm_body,
        grid=(N // MM_BLK,),
        in_specs=[
            pl.BlockSpec((NC, MM_BLK, D), lambda i: (0, i, 0)),
            pl.BlockSpec((D, D), lambda i: (0, 0)),
        ],
        out_specs=pl.BlockSpec((MM_BLK, D), lambda i: (i, 0)),
        out_shape=jax.ShapeDtypeStruct((N, D), jnp.float32),
    )(partials, W)


@jax.jit
def kernel(x, edge_index, W):
    src = edge_index[0].astype(jnp.int32).reshape(NC, NS, NCHUNK, CH)
    dst = edge_index[1].astype(jnp.int32).reshape(NC, NS, NCHUNK, CH)
    partials = _sc_scatter(src, dst, x)
    return _tc_matmul(partials, W)


# speedup vs baseline: 1.1764x; 1.1764x over previous
import functools

import jax
import jax.numpy as jnp
from jax import lax
from jax.experimental import pallas as pl
from jax.experimental.pallas import tpu as pltpu
from jax.experimental.pallas import tpu_sc as plsc

N = 10000
D = 128
E = 320000
NC = 2
NS = 16
NW = NC * NS
CH = 80
E_T = E // NW
NCHUNK = E_T // CH
H0 = 64
H1 = NCHUNK - H0
ROWS_A = 624
TAIL_BASE = ROWS_A * NS
ZR = 24


def _sc_body(src_hbm, dst_hbm, x_hbm, out_hbm,
             src_v, dst_v, rows0, rows1, rows2, stage_v, agg_sh,
             semg0, semg1, semg2, sems0, sems1, sems2):
    cid = lax.axis_index("c")
    sid = lax.axis_index("s")

    z16 = jnp.zeros((16,), jnp.float32)

    @pl.loop(0, ZR)
    def _zero(i):
        for l in range(D // 16):
            stage_v[i, pl.ds(l * 16, 16)] = z16

    base = sid * ROWS_A

    for i in range(ROWS_A // ZR):
        pltpu.async_copy(stage_v, agg_sh.at[pl.ds(base + i * ZR, ZR)], semg0)

    @pl.when(sid == NS - 1)
    def _zero_tail():
        pltpu.async_copy(stage_v.at[pl.ds(0, N - TAIL_BASE)],
                         agg_sh.at[pl.ds(TAIL_BASE, N - TAIL_BASE)], semg0)

    pltpu.async_copy(src_hbm.at[cid, sid, pl.ds(0, H0)], src_v, semg1)
    pltpu.async_copy(dst_hbm.at[cid, sid, pl.ds(0, H0)], dst_v, semg2)

    for i in range(ROWS_A // ZR):
        pltpu.make_async_copy(
            stage_v, agg_sh.at[pl.ds(base + i * ZR, ZR)], semg0).wait()

    @pl.when(sid == NS - 1)
    def _zero_tail_wait():
        pltpu.make_async_copy(
            stage_v.at[pl.ds(0, N - TAIL_BASE)],
            agg_sh.at[pl.ds(TAIL_BASE, N - TAIL_BASE)], semg0).wait()

    pltpu.make_async_copy(
        src_hbm.at[cid, sid, pl.ds(0, H0)], src_v, semg1).wait()
    pltpu.make_async_copy(
        dst_hbm.at[cid, sid, pl.ds(0, H0)], dst_v, semg2).wait()

    plsc.subcore_barrier()

    bufs = [(rows0, semg0, sems0), (rows1, semg1, sems1),
            (rows2, semg2, sems2)]

    def gather(k, r):
        b, sg, _ = bufs[r]
        pltpu.async_copy(x_hbm.at[src_v.at[k]], b, sg)

    def wait_gather(k, r):
        b, sg, _ = bufs[r]
        pltpu.make_async_copy(x_hbm.at[src_v.at[k]], b, sg).wait()

    def scatter(k, r):
        b, _, ss = bufs[r]
        pltpu.async_copy(b, agg_sh.at[dst_v.at[k]], ss, add=True)

    def wait_scatter(k, r):
        b, _, ss = bufs[r]
        pltpu.make_async_copy(b, agg_sh.at[dst_v.at[k]], ss).wait()

    def step(k, r, with_gather=True):
        wait_gather(k, r)
        scatter(k, r)
        wait_scatter(k - 1, (r + 2) % 3)
        if with_gather:
            gather(k + 2, (r + 2) % 3)

    def run_half(M):
        gather(0, 0)
        gather(1, 1)
        wait_gather(0, 0)
        scatter(0, 0)
        gather(2, 2)
        last = 1
        while last + 3 + 4 <= M - 1 + 3:
            last += 3
        last -= 3

        @pl.loop(1, last + 1, step=3)
        def _steady(k):
            step(k, 1)
            step(k + 1, 2)
            step(k + 2, 0)

        for k in range(last + 3, M):
            step(k, k % 3, with_gather=(k + 2 <= M - 1))
        wait_scatter(M - 1, (M - 1) % 3)

    run_half(H0)

    pltpu.async_copy(src_hbm.at[cid, sid, pl.ds(H0, H1)],
                     src_v.at[pl.ds(0, H1)], semg0)
    pltpu.async_copy(dst_hbm.at[cid, sid, pl.ds(H0, H1)],
                     dst_v.at[pl.ds(0, H1)], semg1)
    pltpu.make_async_copy(src_hbm.at[cid, sid, pl.ds(H0, H1)],
                          src_v.at[pl.ds(0, H1)], semg0).wait()
    pltpu.make_async_copy(dst_hbm.at[cid, sid, pl.ds(H0, H1)],
                          dst_v.at[pl.ds(0, H1)], semg1).wait()

    run_half(H1)

    plsc.subcore_barrier()

    sl = pl.ds(base, ROWS_A)
    pltpu.sync_copy(agg_sh.at[sl], out_hbm.at[cid].at[sl])

    @pl.when(sid == NS - 1)
    def _write_tail():
        tl = pl.ds(TAIL_BASE, N - TAIL_BASE)
        pltpu.sync_copy(agg_sh.at[tl], out_hbm.at[cid].at[tl])


_sc_scatter = functools.partial(
    pl.kernel,
    out_type=jax.ShapeDtypeStruct((NC, N, D), jnp.float32),
    mesh=plsc.VectorSubcoreMesh(core_axis_name="c", subcore_axis_name="s"),
    scratch_types=[
        pltpu.VMEM((H0, CH), jnp.int32),
        pltpu.VMEM((H0, CH), jnp.int32),
        pltpu.VMEM((CH, D), jnp.float32),
        pltpu.VMEM((CH, D), jnp.float32),
        pltpu.VMEM((CH, D), jnp.float32),
        pltpu.VMEM((ZR, D), jnp.float32),
        pltpu.VMEM_SHARED((N, D), jnp.float32),
        pltpu.SemaphoreType.DMA,
        pltpu.SemaphoreType.DMA,
        pltpu.SemaphoreType.DMA,
        pltpu.SemaphoreType.DMA,
        pltpu.SemaphoreType.DMA,
        pltpu.SemaphoreType.DMA,
    ],
)(_sc_body)


MM_BLK = 2000


def _mm_body(p_ref, w_ref, o_ref):
    acc = p_ref[0] + p_ref[1]
    o_ref[...] = lax.dot_general(
        acc, w_ref[...], (((1,), (1,)), ((), ())),
        preferred_element_type=jnp.float32)


def _tc_matmul(partials, W):
    return pl.pallas_call(
        _mm_body,
        grid=(N // MM_BLK,),
        in_specs=[
            pl.BlockSpec((NC, MM_BLK, D), lambda i: (0, i, 0)),
            pl.BlockSpec((D, D), lambda i: (0, 0)),
        ],
        out_specs=pl.BlockSpec((MM_BLK, D), lambda i: (i, 0)),
        out_shape=jax.ShapeDtypeStruct((N, D), jnp.float32),
    )(partials, W)


@jax.jit
def kernel(x, edge_index, W):
    src = edge_index[0].astype(jnp.int32).reshape(NC, NS, NCHUNK, CH)
    dst = edge_index[1].astype(jnp.int32).reshape(NC, NS, NCHUNK, CH)
    partials = _sc_scatter(src, dst, x)
    return _tc_matmul(partials, W)
